# baseline (speedup 1.0000x reference)
import jax
import jax.numpy as jnp
from jax.experimental import pallas as pl

F32 = jnp.float32


def _dot(a, b):
    return jnp.dot(a, b, preferred_element_type=F32)


def _xconv(x, w, b, stride=1, pad=1):
    out = jax.lax.conv_general_dilated(
        x, w, (stride, stride), [(pad, pad), (pad, pad)],
        dimension_numbers=("NCHW", "OIHW", "NCHW"))
    return out + b[None, :, None, None]


def _quantize(flat, embed):
    d2 = jnp.sum(flat ** 2, axis=1, keepdims=True)
    e2 = jnp.sum(embed ** 2, axis=0, keepdims=True)
    dist = d2 - 2.0 * _dot(flat, embed) + e2
    mind = jnp.min(dist, axis=1, keepdims=True)
    NE = embed.shape[1]
    iota = jax.lax.broadcasted_iota(jnp.int32, dist.shape, 1)
    ind = jnp.min(jnp.where(dist == mind, iota, NE), axis=1, keepdims=True)
    onehot = (iota == ind).astype(F32)
    q = jnp.dot(onehot, embed.T, preferred_element_type=F32,
                precision=jax.lax.Precision.HIGHEST)
    return flat + (q - flat)


def _conv0_body(xc_ref, w_ref, b_ref, o_ref):
    acc = _dot(xc_ref[...], w_ref[...]) + b_ref[...]
    o_ref[...] = jnp.maximum(acc, 0.0)


def _rtail_body(x_ref, t_ref, w_ref, b_ref, o_ref):
    t = jnp.maximum(t_ref[...], 0.0)
    o_ref[...] = x_ref[...] + (_dot(t, w_ref[...]) + b_ref[...])


def _c3c64_body(x_ref, w_ref, b_ref, o_ref):
    x = x_ref[...]
    B = x.shape[0]
    zr = jnp.zeros((B, 1, 28, 64), F32)
    x = jnp.concatenate([zr, x, zr], axis=1)
    zc = jnp.zeros((B, 30, 1, 64), F32)
    x = jnp.concatenate([zc, x, zc], axis=2)
    parts = [x[:, dh:dh + 28, dw:dw + 28, :].reshape(B * 784, 64)
             for dh in range(3) for dw in range(3)]
    big = jnp.concatenate(parts, axis=1)
    o_ref[...] = _dot(big, w_ref[...]) + b_ref[...]


def _qt_body(x_ref, t_ref, w2_ref, b2_ref, qw_ref, qb_ref, e_ref, o_ref):
    t = jnp.maximum(t_ref[...], 0.0)
    h = x_ref[...] + (_dot(t, w2_ref[...]) + b2_ref[...])
    enc_t = jnp.maximum(h, 0.0)
    qt = _dot(enc_t, qw_ref[...]) + qb_ref[...]
    o_ref[...] = _quantize(qt, e_ref[...])


def _qb_body(d_ref, e_ref, w_ref, b_ref, o_ref):
    cat = jnp.concatenate([d_ref[...], e_ref[...]], axis=1)
    o_ref[...] = _dot(cat, w_ref[...]) + b_ref[...]


def _quantb_body(f_ref, e_ref, o_ref):
    o_ref[...] = _quantize(f_ref[...], e_ref[...])


def _pc(body, out_shape, *args):
    return pl.pallas_call(
        body, out_shape=jax.ShapeDtypeStruct(out_shape, F32))(*args)


def _row(b):
    return b.reshape(1, -1)


def _w3cat(w):
    O, I, _, _ = w.shape
    return w.transpose(2, 3, 1, 0).reshape(9 * I, O)


def _m1x1(w):
    return w[:, :, 0, 0].T


def kernel(x, eb_w0, eb_b0, eb_w1, eb_b1, eb_w2, eb_b2,
           eb_r1w1, eb_r1b1, eb_r1w2, eb_r1b2,
           eb_r2w1, eb_r2b1, eb_r2w2, eb_r2b2,
           et_w0, et_b0, et_w1, et_b1,
           et_r1w1, et_r1b1, et_r1w2, et_r1b2,
           et_r2w1, et_r2b1, et_r2w2, et_r2b2,
           qt_w, qt_b, embed_t,
           dt_w0, dt_b0,
           dt_r1w1, dt_r1b1, dt_r1w2, dt_r1b2,
           dt_r2w1, dt_r2b1, dt_r2w2, dt_r2b2,
           dt_wt, dt_bt,
           qb_w, qb_b, embed_b):
    B = x.shape[0]
    relu = jax.nn.relu

    xp = jnp.pad(x.transpose(0, 2, 3, 1), ((0, 0), (1, 1), (1, 1), (0, 0)))
    cols = [jax.lax.slice(xp, (0, kh, kw, 0), (B, kh + 223, kw + 223, 3), (1, 2, 2, 1))
            for kh in range(4) for kw in range(4)]
    xcol = jnp.concatenate(cols, axis=-1).reshape(B * 112 * 112, 48)
    h1 = _pc(_conv0_body, (B * 112 * 112, 64),
             xcol, eb_w0.transpose(2, 3, 1, 0).reshape(48, 64), _row(eb_b0))
    h1 = h1.reshape(B, 112, 112, 64).transpose(0, 3, 1, 2)

    h2 = relu(_xconv(h1, eb_w1, eb_b1, 2, 1))
    h = _xconv(h2, eb_w2, eb_b2, 1, 1)
    for w1, b1, w2, b2 in ((eb_r1w1, eb_r1b1, eb_r1w2, eb_r1b2),
                           (eb_r2w1, eb_r2b1, eb_r2w2, eb_r2b2)):
        t1 = _xconv(relu(h), w1, b1, 1, 1)
        h2d = h.transpose(0, 2, 3, 1).reshape(B * 3136, 128)
        t2d = t1.transpose(0, 2, 3, 1).reshape(B * 3136, 32)
        h = _pc(_rtail_body, (B * 3136, 128), h2d, t2d, _m1x1(w2), _row(b2))
        h = h.reshape(B, 56, 56, 128).transpose(0, 3, 1, 2)
    enc_b = relu(h)

    h0 = relu(_xconv(enc_b, et_w0, et_b0, 2, 1))
    h = _pc(_c3c64_body, (B * 784, 128),
            h0.transpose(0, 2, 3, 1), _w3cat(et_w1), _row(et_b1))
    h = h.reshape(B, 28, 28, 128).transpose(0, 3, 1, 2)
    t1 = _xconv(relu(h), et_r1w1, et_r1b1, 1, 1)
    h2d = h.transpose(0, 2, 3, 1).reshape(B * 784, 128)
    t2d = t1.transpose(0, 2, 3, 1).reshape(B * 784, 32)
    h = _pc(_rtail_body, (B * 784, 128), h2d, t2d, _m1x1(et_r1w2), _row(et_r1b2))
    h = h.reshape(B, 28, 28, 128).transpose(0, 3, 1, 2)
    t1 = _xconv(relu(h), et_r2w1, et_r2b1, 1, 1)
    h2d = h.transpose(0, 2, 3, 1).reshape(B * 784, 128)
    t2d = t1.transpose(0, 2, 3, 1).reshape(B * 784, 32)
    quant_t2d = _pc(_qt_body, (B * 784, 64),
                    h2d, t2d, _m1x1(et_r2w2), _row(et_r2b2),
                    _m1x1(qt_w), _row(qt_b), embed_t)
    quant_t = quant_t2d.reshape(B, 28, 28, 64)
    quant_tn = quant_t.transpose(0, 3, 1, 2)

    h = _pc(_c3c64_body, (B * 784, 128),
            quant_t, _w3cat(dt_w0), _row(dt_b0))
    h = h.reshape(B, 28, 28, 128).transpose(0, 3, 1, 2)
    for w1, b1, w2, b2 in ((dt_r1w1, dt_r1b1, dt_r1w2, dt_r1b2),
                           (dt_r2w1, dt_r2b1, dt_r2w2, dt_r2b2)):
        t1 = _xconv(relu(h), w1, b1, 1, 1)
        h2d = h.transpose(0, 2, 3, 1).reshape(B * 784, 128)
        t2d = t1.transpose(0, 2, 3, 1).reshape(B * 784, 32)
        h = _pc(_rtail_body, (B * 784, 128), h2d, t2d, _m1x1(w2), _row(b2))
        h = h.reshape(B, 28, 28, 128).transpose(0, 3, 1, 2)
    h = relu(h)
    wt = jnp.flip(dt_wt, (2, 3)).transpose(1, 0, 2, 3)
    dec_t = jax.lax.conv_general_dilated(
        h, wt, (1, 1), [(2, 2), (2, 2)], lhs_dilation=(2, 2),
        dimension_numbers=("NCHW", "OIHW", "NCHW")) + dt_bt[None, :, None, None]

    dec2d = dec_t.transpose(0, 2, 3, 1).reshape(B * 3136, 64)
    enc2d = enc_b.transpose(0, 2, 3, 1).reshape(B * 3136, 128)
    qb = _pc(_qb_body, (B * 3136, 64), dec2d, enc2d, _m1x1(qb_w), _row(qb_b))
    quant_b2d = pl.pallas_call(
        _quantb_body,
        grid=(B,),
        in_specs=[pl.BlockSpec((3136, 64), lambda i: (i, 0)),
                  pl.BlockSpec((64, 512), lambda i: (0, 0))],
        out_specs=pl.BlockSpec((3136, 64), lambda i: (i, 0)),
        out_shape=jax.ShapeDtypeStruct((B * 3136, 64), F32),
    )(qb, embed_b)
    quant_bn = quant_b2d.reshape(B, 56, 56, 64).transpose(0, 3, 1, 2)
    return quant_tn, quant_bn
